# baseline (speedup 1.0000x reference)
import functools

import jax
import jax.numpy as jnp
from jax import lax
from jax.experimental import pallas as pl
from jax.experimental.pallas import tpu as pltpu
from jax.experimental.pallas import tpu_sc as plsc

N = 10000
E = 320000
D = 128
NC = 2
NS = 16
NW = NC * NS
B = 120
NBT = 88
EPT = NBT * B
E_PAD = NW * EPT
DUMMY = N
ACC_N = 10240
ROWS_PT = ACC_N // NS

_mesh = plsc.VectorSubcoreMesh(core_axis_name="c", subcore_axis_name="s")


@functools.partial(
    pl.kernel,
    out_type=jax.ShapeDtypeStruct((NC, ACC_N), jnp.float32),
    mesh=_mesh,
    scratch_types=[
        pltpu.VMEM((NBT, B), jnp.int32),
        pltpu.VMEM((B,), jnp.float32),
        pltpu.VMEM_SHARED((ACC_N,), jnp.float32),
        pltpu.SemaphoreType.DMA,
    ],
)
def _deg_call(dst2_hbm, ones_hbm, z_hbm, out_hbm, dstv, onesb, dacc, sem):
    c = lax.axis_index("c")
    s = lax.axis_index("s")
    wid = c * NS + s
    pltpu.sync_copy(dst2_hbm.at[pl.ds(wid * NBT, NBT)], dstv)
    pltpu.sync_copy(ones_hbm, onesb)
    pltpu.sync_copy(z_hbm, dacc.at[pl.ds(s * ROWS_PT, ROWS_PT)])
    plsc.subcore_barrier()

    def fire(j, carry):
        pltpu.async_copy(onesb, dacc.at[dstv.at[j]], sem, add=True)
        return carry

    lax.fori_loop(0, NBT, fire, 0)

    def drain(j, carry):
        pltpu.make_async_copy(onesb, dacc.at[dstv.at[j]], sem).wait()
        return carry

    lax.fori_loop(0, NBT, drain, 0)
    plsc.subcore_barrier()
    pltpu.sync_copy(dacc.at[pl.ds(s * ROWS_PT, ROWS_PT)],
                    out_hbm.at[c, pl.ds(s * ROWS_PT, ROWS_PT)])


_R = 1024


def _hs_body(x_ref, w_ref, degp_ref, hs_ref):
    i = pl.program_id(0)
    deg = (degp_ref[0, pl.ds(i * _R, _R)] + degp_ref[1, pl.ds(i * _R, _R)]
           + 1.0)
    dis = lax.rsqrt(deg)
    h = jnp.dot(x_ref[...], w_ref[...], preferred_element_type=jnp.float32)
    hs_ref[...] = h * dis[:, None]


def _hs_call(x, W, degp):
    return pl.pallas_call(
        _hs_body,
        grid=(ACC_N // _R,),
        in_specs=[
            pl.BlockSpec((_R, D), lambda i: (i, 0)),
            pl.BlockSpec((D, D), lambda i: (0, 0)),
            pl.BlockSpec((NC, ACC_N), lambda i: (0, 0)),
        ],
        out_specs=pl.BlockSpec((_R, D), lambda i: (i, 0)),
        out_shape=jax.ShapeDtypeStruct((ACC_N, D), jnp.float32),
    )(x, W, degp)


@functools.partial(
    pl.kernel,
    out_type=jax.ShapeDtypeStruct((NC, ACC_N, D), jnp.float32),
    mesh=_mesh,
    scratch_types=[
        pltpu.VMEM((B, D), jnp.float32),
        pltpu.VMEM((B, D), jnp.float32),
        pltpu.VMEM((NBT, B), jnp.int32),
        pltpu.VMEM((2, B), jnp.int32),
        pltpu.VMEM_SHARED((ACC_N, D), jnp.float32),
        pltpu.SemaphoreType.DMA,
        pltpu.SemaphoreType.DMA,
        pltpu.SemaphoreType.DMA,
        pltpu.SemaphoreType.DMA,
    ],
)
def _agg_call(hs_hbm, src_hbm, dst_hbm, z_hbm, out_hbm,
              buf0, buf1, srcv, dstr, acc, g0, g1, d0, d1):
    c = lax.axis_index("c")
    s = lax.axis_index("s")
    wid = c * NS + s
    row0 = wid * NBT
    pltpu.sync_copy(src_hbm.at[pl.ds(row0, NBT)], srcv)
    pltpu.sync_copy(z_hbm, acc.at[pl.ds(s * ROWS_PT, ROWS_PT)])
    plsc.subcore_barrier()

    dsems = (d0, d1)
    gsems = (g0, g1)
    bufs = (buf0, buf1)

    def _dfetch(j, slot):
        pltpu.async_copy(dst_hbm.at[row0 + j, 0], dstr.at[slot], dsems[slot])

    def _dwait(j, slot):
        pltpu.make_async_copy(dst_hbm.at[row0 + j, 0], dstr.at[slot],
                              dsems[slot]).wait()

    def _scatter(slot):
        pltpu.sync_copy(bufs[slot], acc.at[dstr.at[slot]], add=True)

    def _gstart(j, slot):
        pltpu.async_copy(hs_hbm.at[srcv.at[j]], bufs[slot], gsems[slot])

    def _gwait(j, slot):
        pltpu.make_async_copy(hs_hbm.at[srcv.at[j]], bufs[slot],
                              gsems[slot]).wait()

    _dfetch(0, 0)
    _gstart(0, 0)
    _dfetch(1, 1)
    _gstart(1, 1)

    def body(jj, carry):
        j0 = jj * 2
        for slot in (0, 1):
            j = j0 + slot
            _gwait(j, slot)
            _dwait(j, slot)
            _scatter(slot)

            @pl.when(j + 2 < NBT)
            def _():
                _dfetch(j + 2, slot)
                _gstart(j + 2, slot)

        return carry

    lax.fori_loop(0, NBT // 2, body, 0)
    plsc.subcore_barrier()
    pltpu.sync_copy(acc.at[pl.ds(s * ROWS_PT, ROWS_PT)],
                    out_hbm.at[c, pl.ds(s * ROWS_PT, ROWS_PT)])


def _final_body(acc_ref, hs_ref, degp_ref, b_ref, out_ref):
    i = pl.program_id(0)
    deg = (degp_ref[0, pl.ds(i * _R, _R)] + degp_ref[1, pl.ds(i * _R, _R)]
           + 1.0)
    dis = lax.rsqrt(deg)
    a = acc_ref[0] + acc_ref[1] + hs_ref[...]
    o = a * dis[:, None] + b_ref[...]
    m = jnp.max(o, axis=1, keepdims=True)
    lse = jnp.log(jnp.sum(jnp.exp(o - m), axis=1, keepdims=True)) + m
    out_ref[...] = o - lse


def _final_call(acc, hs, degp, b2):
    return pl.pallas_call(
        _final_body,
        grid=(ACC_N // _R,),
        in_specs=[
            pl.BlockSpec((NC, _R, D), lambda i: (0, i, 0)),
            pl.BlockSpec((_R, D), lambda i: (i, 0)),
            pl.BlockSpec((NC, ACC_N), lambda i: (0, 0)),
            pl.BlockSpec((1, D), lambda i: (0, 0)),
        ],
        out_specs=pl.BlockSpec((_R, D), lambda i: (i, 0)),
        out_shape=jax.ShapeDtypeStruct((ACC_N, D), jnp.float32),
    )(acc, hs, degp, b2)


def kernel(x, edge_index, W, b):
    x_p = jnp.concatenate(
        [x, jnp.zeros((ACC_N - N, D), jnp.float32)], axis=0)
    pad = E_PAD - E
    src_p = jnp.concatenate([edge_index[0],
                             jnp.zeros((pad,), jnp.int32)])
    dst_p = jnp.concatenate(
        [edge_index[1],
         DUMMY + (jnp.arange(pad, dtype=jnp.int32) % (ACC_N - N))])
    src2 = src_p.reshape(E_PAD // B, B)
    dst2 = dst_p.reshape(E_PAD // B, B)
    dst3 = dst_p.reshape(E_PAD // B, 1, B)
    degp = _deg_call(dst2,
                     jnp.ones((B,), jnp.float32),
                     jnp.zeros((ROWS_PT,), jnp.float32))
    hs = _hs_call(x_p, W, degp)
    acc = _agg_call(hs, src2, dst3,
                    jnp.zeros((ROWS_PT, D), jnp.float32))
    return _final_call(acc, hs, degp, b.reshape(1, D))[:N]


# speedup vs baseline: 6.2955x; 6.2955x over previous
import functools

import jax
import jax.numpy as jnp
from jax import lax
from jax.experimental import pallas as pl
from jax.experimental.pallas import tpu as pltpu
from jax.experimental.pallas import tpu_sc as plsc

N = 10000
E = 320000
D = 128
HD = D // 2
NC = 2
NS = 16
NW = NC * NS
B = 128
DEG_NBT = 80
AGG_NBT = 160
E_PAD = NW * DEG_NBT * B
ACC_N = 10240
ROWS_PT = ACC_N // NS
DUMMY = N

_mesh = plsc.VectorSubcoreMesh(core_axis_name="c", subcore_axis_name="s")


@functools.partial(
    pl.kernel,
    out_type=jax.ShapeDtypeStruct((NC, ACC_N), jnp.float32),
    mesh=_mesh,
    scratch_types=[
        pltpu.VMEM((DEG_NBT, B), jnp.int32),
        pltpu.VMEM((B,), jnp.float32),
        pltpu.VMEM_SHARED((ACC_N,), jnp.float32),
        pltpu.SemaphoreType.DMA,
    ],
)
def _deg_call(dst2_hbm, ones_hbm, z_hbm, out_hbm, dstv, onesb, dacc, sem):
    c = lax.axis_index("c")
    s = lax.axis_index("s")
    wid = c * NS + s
    pltpu.sync_copy(dst2_hbm.at[pl.ds(wid * DEG_NBT, DEG_NBT)], dstv)
    pltpu.sync_copy(ones_hbm, onesb)
    pltpu.sync_copy(z_hbm, dacc.at[pl.ds(s * ROWS_PT, ROWS_PT)])
    plsc.subcore_barrier()

    def fire(j, carry):
        pltpu.async_copy(onesb, dacc.at[dstv.at[j]], sem, add=True)
        return carry

    lax.fori_loop(0, DEG_NBT, fire, 0)

    def drain(j, carry):
        pltpu.make_async_copy(onesb, dacc.at[dstv.at[j]], sem).wait()
        return carry

    lax.fori_loop(0, DEG_NBT, drain, 0)
    plsc.subcore_barrier()
    pltpu.sync_copy(dacc.at[pl.ds(s * ROWS_PT, ROWS_PT)],
                    out_hbm.at[c, pl.ds(s * ROWS_PT, ROWS_PT)])


_R = 1024


def _hs_body(x_ref, w_ref, degp_ref, hs_ref, hs2_ref):
    i = pl.program_id(0)
    deg = (degp_ref[0, pl.ds(i * _R, _R)] + degp_ref[1, pl.ds(i * _R, _R)]
           + 1.0)
    dis = lax.rsqrt(deg)
    h = jnp.dot(x_ref[...], w_ref[...], preferred_element_type=jnp.float32)
    hsv = h * dis[:, None]
    hs_ref[...] = hsv
    hs2_ref[0] = hsv[:, :HD]
    hs2_ref[1] = hsv[:, HD:]


def _hs_call(x, W, degp):
    return pl.pallas_call(
        _hs_body,
        grid=(ACC_N // _R,),
        in_specs=[
            pl.BlockSpec((_R, D), lambda i: (i, 0)),
            pl.BlockSpec((D, D), lambda i: (0, 0)),
            pl.BlockSpec((NC, ACC_N), lambda i: (0, 0)),
        ],
        out_specs=[
            pl.BlockSpec((_R, D), lambda i: (i, 0)),
            pl.BlockSpec((NC, _R, HD), lambda i: (0, i, 0)),
        ],
        out_shape=[
            jax.ShapeDtypeStruct((ACC_N, D), jnp.float32),
            jax.ShapeDtypeStruct((NC, ACC_N, HD), jnp.float32),
        ],
    )(x, W, degp)


@functools.partial(
    pl.kernel,
    out_type=jax.ShapeDtypeStruct((NC, ACC_N, HD), jnp.float32),
    mesh=_mesh,
    scratch_types=[
        pltpu.VMEM((B, HD), jnp.float32),
        pltpu.VMEM((B, HD), jnp.float32),
        pltpu.VMEM((B, HD), jnp.float32),
        pltpu.VMEM((B, HD), jnp.float32),
        pltpu.VMEM((AGG_NBT, B), jnp.int32),
        pltpu.VMEM((AGG_NBT, B), jnp.int32),
        pltpu.VMEM_SHARED((ACC_N, HD), jnp.float32),
        pltpu.SemaphoreType.DMA,
        pltpu.SemaphoreType.DMA,
        pltpu.SemaphoreType.DMA,
        pltpu.SemaphoreType.DMA,
        pltpu.SemaphoreType.DMA,
        pltpu.SemaphoreType.DMA,
        pltpu.SemaphoreType.DMA,
        pltpu.SemaphoreType.DMA,
    ],
    compiler_params=pltpu.CompilerParams(use_tc_tiling_on_sc=False),
)
def _agg_call(hs2_hbm, src_hbm, dst_hbm, z_hbm, out_hbm,
              buf0, buf1, buf2, buf3, srcv, dstv, acc,
              g0, g1, g2, g3, s0, s1, s2, s3):
    c = lax.axis_index("c")
    s = lax.axis_index("s")
    row0 = s * AGG_NBT
    pltpu.sync_copy(src_hbm.at[pl.ds(row0, AGG_NBT)], srcv)
    pltpu.sync_copy(dst_hbm.at[pl.ds(row0, AGG_NBT)], dstv)
    pltpu.sync_copy(z_hbm, acc.at[pl.ds(s * ROWS_PT, ROWS_PT)])
    plsc.subcore_barrier()

    gsems = (g0, g1, g2, g3)
    ssems = (s0, s1, s2, s3)
    bufs = (buf0, buf1, buf2, buf3)

    def run(hs_c):
        def _gstart(j, slot):
            pltpu.async_copy(hs_c.at[srcv.at[j]], bufs[slot], gsems[slot])

        def _gwait(j, slot):
            pltpu.make_async_copy(hs_c.at[srcv.at[j]], bufs[slot],
                                  gsems[slot]).wait()

        def _sstart(j, slot):
            pltpu.async_copy(bufs[slot], acc.at[dstv.at[j]], ssems[slot],
                             add=True)

        def _swait(j, slot):
            pltpu.make_async_copy(bufs[slot], acc.at[dstv.at[j]],
                                  ssems[slot]).wait()

        for j in (0, 1, 2):
            _gstart(j, j)

        def body(jj, carry):
            for u in range(4):
                j = jj * 4 + u
                slot = u
                wslot = (u + 3) % 4
                _gwait(j, slot)
                _sstart(j, slot)

                @pl.when(j >= 1)
                def _():
                    _swait(j - 1, wslot)

                @pl.when(j + 3 < AGG_NBT)
                def _():
                    _gstart(j + 3, wslot)

            return carry

        lax.fori_loop(0, AGG_NBT // 4, body, 0)
        _swait(AGG_NBT - 1, (AGG_NBT - 1) % 4)

    @pl.when(c == 0)
    def _():
        run(hs2_hbm.at[0])

    @pl.when(c == 1)
    def _():
        run(hs2_hbm.at[1])

    plsc.subcore_barrier()
    pltpu.sync_copy(acc.at[pl.ds(s * ROWS_PT, ROWS_PT)],
                    out_hbm.at[c, pl.ds(s * ROWS_PT, ROWS_PT)])


def _final_body(acc_ref, hs_ref, degp_ref, b_ref, out_ref):
    i = pl.program_id(0)
    deg = (degp_ref[0, pl.ds(i * _R, _R)] + degp_ref[1, pl.ds(i * _R, _R)]
           + 1.0)
    dis = lax.rsqrt(deg)
    a = jnp.concatenate(
        [acc_ref[0] + hs_ref[:, :HD], acc_ref[1] + hs_ref[:, HD:]], axis=1)
    o = a * dis[:, None] + b_ref[...]
    m = jnp.max(o, axis=1, keepdims=True)
    lse = jnp.log(jnp.sum(jnp.exp(o - m), axis=1, keepdims=True)) + m
    out_ref[...] = o - lse


def _final_call(acc, hs, degp, b2):
    return pl.pallas_call(
        _final_body,
        grid=(ACC_N // _R,),
        in_specs=[
            pl.BlockSpec((NC, _R, HD), lambda i: (0, i, 0)),
            pl.BlockSpec((_R, D), lambda i: (i, 0)),
            pl.BlockSpec((NC, ACC_N), lambda i: (0, 0)),
            pl.BlockSpec((1, D), lambda i: (0, 0)),
        ],
        out_specs=pl.BlockSpec((_R, D), lambda i: (i, 0)),
        out_shape=jax.ShapeDtypeStruct((ACC_N, D), jnp.float32),
    )(acc, hs, degp, b2)


def kernel(x, edge_index, W, b):
    x_p = jnp.concatenate(
        [x, jnp.zeros((ACC_N - N, D), jnp.float32)], axis=0)
    pad = E_PAD - E
    src_p = jnp.concatenate([edge_index[0],
                             jnp.arange(pad, dtype=jnp.int32) % N])
    dst_p = jnp.concatenate(
        [edge_index[1],
         DUMMY + (jnp.arange(pad, dtype=jnp.int32) % (ACC_N - N))])
    src2 = src_p.reshape(E_PAD // B, B)
    dst2 = dst_p.reshape(E_PAD // B, B)
    degp = _deg_call(dst2,
                     jnp.ones((B,), jnp.float32),
                     jnp.zeros((ROWS_PT,), jnp.float32))
    hs, hs2 = _hs_call(x_p, W, degp)
    acc = _agg_call(hs2, src2, dst2,
                    jnp.zeros((ROWS_PT, HD), jnp.float32))
    return _final_call(acc, hs, degp, b.reshape(1, D))[:N]


# speedup vs baseline: 6.8009x; 1.0803x over previous
import functools

import jax
import jax.numpy as jnp
from jax import lax
from jax.experimental import pallas as pl
from jax.experimental.pallas import tpu as pltpu
from jax.experimental.pallas import tpu_sc as plsc

N = 10000
E = 320000
D = 128
NC = 2
NS = 16
NW = NC * NS
B = 125
NBT = 80
EPT = NBT * B
E_PAD = NW * EPT
DUMMY = N
ACC_N = 10240
ROWS_PT = ACC_N // NS

_mesh = plsc.VectorSubcoreMesh(core_axis_name="c", subcore_axis_name="s")


@functools.partial(
    pl.kernel,
    out_type=jax.ShapeDtypeStruct((NC, ACC_N), jnp.float32),
    mesh=_mesh,
    scratch_types=[
        pltpu.VMEM((NBT, B), jnp.int32),
        pltpu.VMEM((B,), jnp.float32),
        pltpu.VMEM_SHARED((ACC_N,), jnp.float32),
        pltpu.SemaphoreType.DMA,
    ],
)
def _deg_call(dst2_hbm, ones_hbm, z_hbm, out_hbm, dstv, onesb, dacc, sem):
    c = lax.axis_index("c")
    s = lax.axis_index("s")
    wid = c * NS + s
    pltpu.sync_copy(dst2_hbm.at[pl.ds(wid * NBT, NBT)], dstv)
    pltpu.sync_copy(ones_hbm, onesb)
    pltpu.sync_copy(z_hbm, dacc.at[pl.ds(s * ROWS_PT, ROWS_PT)])
    plsc.subcore_barrier()

    def fire(j, carry):
        pltpu.async_copy(onesb, dacc.at[dstv.at[j]], sem, add=True)
        return carry

    lax.fori_loop(0, NBT, fire, 0)

    def drain(j, carry):
        pltpu.make_async_copy(onesb, dacc.at[dstv.at[j]], sem).wait()
        return carry

    lax.fori_loop(0, NBT, drain, 0)
    plsc.subcore_barrier()
    pltpu.sync_copy(dacc.at[pl.ds(s * ROWS_PT, ROWS_PT)],
                    out_hbm.at[c, pl.ds(s * ROWS_PT, ROWS_PT)])


_R = 1024


def _hs_body(x_ref, w_ref, degp_ref, hs_ref):
    i = pl.program_id(0)
    deg = (degp_ref[0, pl.ds(i * _R, _R)] + degp_ref[1, pl.ds(i * _R, _R)]
           + 1.0)
    dis = lax.rsqrt(deg)
    h = jnp.dot(x_ref[...], w_ref[...], preferred_element_type=jnp.float32)
    hs_ref[...] = h * dis[:, None]


def _hs_call(x, W, degp):
    return pl.pallas_call(
        _hs_body,
        grid=(ACC_N // _R,),
        in_specs=[
            pl.BlockSpec((_R, D), lambda i: (i, 0)),
            pl.BlockSpec((D, D), lambda i: (0, 0)),
            pl.BlockSpec((NC, ACC_N), lambda i: (0, 0)),
        ],
        out_specs=pl.BlockSpec((_R, D), lambda i: (i, 0)),
        out_shape=jax.ShapeDtypeStruct((ACC_N, D), jnp.float32),
    )(x, W, degp)


@functools.partial(
    pl.kernel,
    out_type=jax.ShapeDtypeStruct((NC, ACC_N, D), jnp.float32),
    mesh=_mesh,
    scratch_types=[
        pltpu.VMEM((B, D), jnp.float32),
        pltpu.VMEM((B, D), jnp.float32),
        pltpu.VMEM((NBT, B), jnp.int32),
        pltpu.VMEM((2, B), jnp.int32),
        pltpu.VMEM_SHARED((ACC_N, D), jnp.float32),
        pltpu.SemaphoreType.DMA,
        pltpu.SemaphoreType.DMA,
        pltpu.SemaphoreType.DMA,
        pltpu.SemaphoreType.DMA,
    ],
)
def _agg_call(hs_hbm, src_hbm, dst_hbm, z_hbm, out_hbm,
              buf0, buf1, srcv, dstr, acc, g0, g1, d0, d1):
    c = lax.axis_index("c")
    s = lax.axis_index("s")
    wid = c * NS + s
    row0 = wid * NBT
    pltpu.sync_copy(src_hbm.at[pl.ds(row0, NBT)], srcv)
    pltpu.sync_copy(z_hbm, acc.at[pl.ds(s * ROWS_PT, ROWS_PT)])
    plsc.subcore_barrier()

    dsems = (d0, d1)
    gsems = (g0, g1)
    bufs = (buf0, buf1)

    def _dfetch(j, slot):
        pltpu.async_copy(dst_hbm.at[row0 + j, 0], dstr.at[slot], dsems[slot])

    def _dwait(j, slot):
        pltpu.make_async_copy(dst_hbm.at[row0 + j, 0], dstr.at[slot],
                              dsems[slot]).wait()

    def _scatter(slot):
        pltpu.sync_copy(bufs[slot], acc.at[dstr.at[slot]], add=True)

    def _gstart(j, slot):
        pltpu.async_copy(hs_hbm.at[srcv.at[j]], bufs[slot], gsems[slot])

    def _gwait(j, slot):
        pltpu.make_async_copy(hs_hbm.at[srcv.at[j]], bufs[slot],
                              gsems[slot]).wait()

    _dfetch(0, 0)
    _gstart(0, 0)
    _dfetch(1, 1)
    _gstart(1, 1)

    def body(jj, carry):
        j0 = jj * 2
        for slot in (0, 1):
            j = j0 + slot
            _gwait(j, slot)
            _dwait(j, slot)
            _scatter(slot)

            @pl.when(j + 2 < NBT)
            def _():
                _dfetch(j + 2, slot)
                _gstart(j + 2, slot)

        return carry

    lax.fori_loop(0, NBT // 2, body, 0)
    plsc.subcore_barrier()
    pltpu.sync_copy(acc.at[pl.ds(s * ROWS_PT, ROWS_PT)],
                    out_hbm.at[c, pl.ds(s * ROWS_PT, ROWS_PT)])


def _final_body(acc_ref, hs_ref, degp_ref, b_ref, out_ref):
    i = pl.program_id(0)
    deg = (degp_ref[0, pl.ds(i * _R, _R)] + degp_ref[1, pl.ds(i * _R, _R)]
           + 1.0)
    dis = lax.rsqrt(deg)
    a = acc_ref[0] + acc_ref[1] + hs_ref[...]
    o = a * dis[:, None] + b_ref[...]
    m = jnp.max(o, axis=1, keepdims=True)
    lse = jnp.log(jnp.sum(jnp.exp(o - m), axis=1, keepdims=True)) + m
    out_ref[...] = o - lse


def _final_call(acc, hs, degp, b2):
    return pl.pallas_call(
        _final_body,
        grid=(ACC_N // _R,),
        in_specs=[
            pl.BlockSpec((NC, _R, D), lambda i: (0, i, 0)),
            pl.BlockSpec((_R, D), lambda i: (i, 0)),
            pl.BlockSpec((NC, ACC_N), lambda i: (0, 0)),
            pl.BlockSpec((1, D), lambda i: (0, 0)),
        ],
        out_specs=pl.BlockSpec((_R, D), lambda i: (i, 0)),
        out_shape=jax.ShapeDtypeStruct((ACC_N, D), jnp.float32),
    )(acc, hs, degp, b2)


def kernel(x, edge_index, W, b):
    x_p = jnp.concatenate(
        [x, jnp.zeros((ACC_N - N, D), jnp.float32)], axis=0)
    pad = E_PAD - E
    src_p = jnp.concatenate([edge_index[0],
                             jnp.zeros((pad,), jnp.int32)])
    dst_p = jnp.concatenate(
        [edge_index[1],
         DUMMY + (jnp.arange(pad, dtype=jnp.int32) % (ACC_N - N))])
    src2 = src_p.reshape(E_PAD // B, B)
    dst2 = dst_p.reshape(E_PAD // B, B)
    dst3 = dst_p.reshape(E_PAD // B, 1, B)
    degp = _deg_call(dst2,
                     jnp.ones((B,), jnp.float32),
                     jnp.zeros((ROWS_PT,), jnp.float32))
    hs = _hs_call(x_p, W, degp)
    acc = _agg_call(hs, src2, dst3,
                    jnp.zeros((ROWS_PT, D), jnp.float32))
    return _final_call(acc, hs, degp, b.reshape(1, D))[:N]


# speedup vs baseline: 6.8049x; 1.0006x over previous
import functools

import jax
import jax.numpy as jnp
from jax import lax
from jax.experimental import pallas as pl
from jax.experimental.pallas import tpu as pltpu
from jax.experimental.pallas import tpu_sc as plsc

N = 10000
E = 320000
D = 128
NC = 2
NS = 16
NW = NC * NS
B = 125
NBT = 80
EPT = NBT * B
ACC_N = 10240
ROWS_PT = ACC_N // NS

_mesh = plsc.VectorSubcoreMesh(core_axis_name="c", subcore_axis_name="s")


@functools.partial(
    pl.kernel,
    out_type=jax.ShapeDtypeStruct((NC, ACC_N), jnp.float32),
    mesh=_mesh,
    scratch_types=[
        pltpu.VMEM((NBT, B), jnp.int32),
        pltpu.VMEM((B,), jnp.float32),
        pltpu.VMEM_SHARED((ACC_N,), jnp.float32),
        pltpu.SemaphoreType.DMA,
    ],
)
def _deg_call(dst2_hbm, ones_hbm, z_hbm, out_hbm, dstv, onesb, dacc, sem):
    c = lax.axis_index("c")
    s = lax.axis_index("s")
    wid = c * NS + s
    pltpu.sync_copy(dst2_hbm.at[pl.ds(wid * NBT, NBT)], dstv)
    pltpu.sync_copy(ones_hbm, onesb)
    pltpu.sync_copy(z_hbm, dacc.at[pl.ds(s * ROWS_PT, ROWS_PT)])
    plsc.subcore_barrier()

    def fire(j, carry):
        pltpu.async_copy(onesb, dacc.at[dstv.at[j]], sem, add=True)
        return carry

    lax.fori_loop(0, NBT, fire, 0)

    def drain(j, carry):
        pltpu.make_async_copy(onesb, dacc.at[dstv.at[j]], sem).wait()
        return carry

    lax.fori_loop(0, NBT, drain, 0)
    plsc.subcore_barrier()
    pltpu.sync_copy(dacc.at[pl.ds(s * ROWS_PT, ROWS_PT)],
                    out_hbm.at[c, pl.ds(s * ROWS_PT, ROWS_PT)])


_R = 1024


def _hs_body(x_ref, w_ref, degp_ref, hs_ref):
    i = pl.program_id(0)
    deg = (degp_ref[0, pl.ds(i * _R, _R)] + degp_ref[1, pl.ds(i * _R, _R)]
           + 1.0)
    dis = lax.rsqrt(deg)
    h = jnp.dot(x_ref[...], w_ref[...], preferred_element_type=jnp.float32)
    hs_ref[...] = h * dis[:, None]


def _hs_call(x, W, degp):
    return pl.pallas_call(
        _hs_body,
        grid=(ACC_N // _R,),
        in_specs=[
            pl.BlockSpec((_R, D), lambda i: (i, 0)),
            pl.BlockSpec((D, D), lambda i: (0, 0)),
            pl.BlockSpec((NC, ACC_N), lambda i: (0, 0)),
        ],
        out_specs=pl.BlockSpec((_R, D), lambda i: (i, 0)),
        out_shape=jax.ShapeDtypeStruct((ACC_N, D), jnp.float32),
    )(x, W, degp)


@functools.partial(
    pl.kernel,
    out_type=jax.ShapeDtypeStruct((NC, ACC_N, D), jnp.float32),
    mesh=_mesh,
    scratch_types=[
        pltpu.VMEM((B, D), jnp.float32),
        pltpu.VMEM((B, D), jnp.float32),
        pltpu.VMEM((NBT, B), jnp.int32),
        pltpu.VMEM((2, B), jnp.int32),
        pltpu.VMEM_SHARED((ACC_N, D), jnp.float32),
        pltpu.SemaphoreType.DMA,
        pltpu.SemaphoreType.DMA,
        pltpu.SemaphoreType.DMA,
        pltpu.SemaphoreType.DMA,
    ],
)
def _agg_call(hs_hbm, src_hbm, dst_hbm, z_hbm, out_hbm,
              buf0, buf1, srcv, dstr, acc, g0, g1, d0, d1):
    c = lax.axis_index("c")
    s = lax.axis_index("s")
    wid = c * NS + s
    row0 = wid * NBT
    pltpu.sync_copy(src_hbm.at[pl.ds(row0, NBT)], srcv)
    pltpu.sync_copy(z_hbm, acc.at[pl.ds(s * ROWS_PT, ROWS_PT)])
    plsc.subcore_barrier()

    dsems = (d0, d1)
    gsems = (g0, g1)
    bufs = (buf0, buf1)

    def _dfetch(j, slot):
        pltpu.async_copy(dst_hbm.at[row0 + j, 0], dstr.at[slot], dsems[slot])

    def _dwait(j, slot):
        pltpu.make_async_copy(dst_hbm.at[row0 + j, 0], dstr.at[slot],
                              dsems[slot]).wait()

    def _scatter(slot):
        pltpu.sync_copy(bufs[slot], acc.at[dstr.at[slot]], add=True)

    def _gstart(j, slot):
        pltpu.async_copy(hs_hbm.at[srcv.at[j]], bufs[slot], gsems[slot])

    def _gwait(j, slot):
        pltpu.make_async_copy(hs_hbm.at[srcv.at[j]], bufs[slot],
                              gsems[slot]).wait()

    _dfetch(0, 0)
    _gstart(0, 0)
    _dfetch(1, 1)
    _gstart(1, 1)

    def body(jj, carry):
        j0 = jj * 2
        for slot in (0, 1):
            j = j0 + slot
            _gwait(j, slot)
            _dwait(j, slot)
            _scatter(slot)

            @pl.when(j + 2 < NBT)
            def _():
                _dfetch(j + 2, slot)
                _gstart(j + 2, slot)

        return carry

    lax.fori_loop(0, NBT // 2, body, 0)
    plsc.subcore_barrier()
    pltpu.sync_copy(acc.at[pl.ds(s * ROWS_PT, ROWS_PT)],
                    out_hbm.at[c, pl.ds(s * ROWS_PT, ROWS_PT)])


def _final_body(acc_ref, hs_ref, degp_ref, b_ref, out_ref):
    i = pl.program_id(0)
    deg = (degp_ref[0, pl.ds(i * _R, _R)] + degp_ref[1, pl.ds(i * _R, _R)]
           + 1.0)
    dis = lax.rsqrt(deg)
    a = acc_ref[0] + acc_ref[1] + hs_ref[...]
    o = a * dis[:, None] + b_ref[...]
    m = jnp.max(o, axis=1, keepdims=True)
    lse = jnp.log(jnp.sum(jnp.exp(o - m), axis=1, keepdims=True)) + m
    out_ref[...] = o - lse


def _final_call(acc, hs, degp, b2):
    return pl.pallas_call(
        _final_body,
        grid=(ACC_N // _R,),
        in_specs=[
            pl.BlockSpec((NC, _R, D), lambda i: (0, i, 0)),
            pl.BlockSpec((_R, D), lambda i: (i, 0)),
            pl.BlockSpec((NC, ACC_N), lambda i: (0, 0)),
            pl.BlockSpec((1, D), lambda i: (0, 0)),
        ],
        out_specs=pl.BlockSpec((_R, D), lambda i: (i, 0)),
        out_shape=jax.ShapeDtypeStruct((ACC_N, D), jnp.float32),
    )(acc, hs, degp, b2)


def kernel(x, edge_index, W, b):
    x_p = jnp.concatenate(
        [x, jnp.zeros((ACC_N - N, D), jnp.float32)], axis=0)
    src2 = edge_index[0].reshape(E // B, B)
    dst2 = edge_index[1].reshape(E // B, B)
    dst3 = edge_index[1].reshape(E // B, 1, B)
    degp = _deg_call(dst2,
                     jnp.ones((B,), jnp.float32),
                     jnp.zeros((ROWS_PT,), jnp.float32))
    hs = _hs_call(x_p, W, degp)
    acc = _agg_call(hs, src2, dst3,
                    jnp.zeros((ROWS_PT, D), jnp.float32))
    return _final_call(acc, hs, degp, b.reshape(1, D))[:N]
